# Initial kernel scaffold; baseline (speedup 1.0000x reference)
import jax
import jax.numpy as jnp
from jax.experimental import pallas as pl


def kernel(hidden_states, expert_routing_table, router_weights, w13_weight, w2_weight):
    raise NotImplementedError("write your pallas kernel here")


# speedup vs baseline: 1.1019x; 1.1019x over previous
import jax
import jax.numpy as jnp
from jax.experimental import pallas as pl
from jax.experimental.pallas import tpu as pltpu

BT = 256
E = 8
D = 2048
I = 2048
TN = 512
NT = I // TN
TND = 512
ND = D // TND


def _k1_body(x_ref, wu_ref, wg_ref, h_ref):
    x = x_ref[...]
    u = jax.lax.dot_general(x, wu_ref[0], (((1,), (1,)), ((), ())),
                            preferred_element_type=jnp.float32)
    g = jax.lax.dot_general(x, wg_ref[0], (((1,), (1,)), ((), ())),
                            preferred_element_type=jnp.float32)
    h_ref[0] = (u * jax.nn.sigmoid(u)) * g


def _k2_body(ert_ref, rw_ref, h_ref, w2_ref, y_ref):
    e = pl.program_id(0)
    h = h_ref[0]
    y = jax.lax.dot_general(h, w2_ref[0], (((1,), (1,)), ((), ())),
                            preferred_element_type=jnp.float32)
    sel = (ert_ref[...] == e).astype(jnp.float32) * rw_ref[...]
    we = jnp.sum(sel, axis=1, keepdims=True)
    y_ref[0] = y * we


def _k3_body(y_ref, o_ref):
    e = pl.program_id(0)

    @pl.when(e == 0)
    def _():
        o_ref[...] = jnp.zeros_like(o_ref)

    o_ref[...] += y_ref[0]


def kernel(hidden_states, expert_routing_table, router_weights, w13_weight, w2_weight):
    x = hidden_states.astype(jnp.float32)
    ert = expert_routing_table.astype(jnp.int32)
    rw = router_weights.astype(jnp.float32)

    h = pl.pallas_call(
        _k1_body,
        grid=(E, NT),
        in_specs=[
            pl.BlockSpec((BT, D), lambda e, n: (0, 0)),
            pl.BlockSpec((1, TN, D), lambda e, n: (e, n, 0)),
            pl.BlockSpec((1, TN, D), lambda e, n: (e, n + NT, 0)),
        ],
        out_specs=pl.BlockSpec((1, BT, TN), lambda e, n: (e, 0, n)),
        out_shape=jax.ShapeDtypeStruct((E, BT, I), jnp.float32),
    )(x, w13_weight, w13_weight)

    yw = pl.pallas_call(
        _k2_body,
        grid=(E, ND),
        in_specs=[
            pl.BlockSpec((BT, 2), lambda e, n: (0, 0)),
            pl.BlockSpec((BT, 2), lambda e, n: (0, 0)),
            pl.BlockSpec((1, BT, I), lambda e, n: (e, 0, 0)),
            pl.BlockSpec((1, TND, I), lambda e, n: (e, n, 0)),
        ],
        out_specs=pl.BlockSpec((1, BT, TND), lambda e, n: (e, 0, n)),
        out_shape=jax.ShapeDtypeStruct((E, BT, D), jnp.float32),
    )(ert, rw, h, w2_weight)

    out = pl.pallas_call(
        _k3_body,
        grid=(E,),
        in_specs=[pl.BlockSpec((1, BT, D), lambda e: (e, 0, 0))],
        out_specs=pl.BlockSpec((BT, D), lambda e: (0, 0)),
        out_shape=jax.ShapeDtypeStruct((BT, D), jnp.float32),
    )(yw)
    return out
